# baseline (speedup 1.0000x reference)
import functools

import jax
import jax.numpy as jnp
from jax import lax
from jax.experimental import pallas as pl
from jax.experimental.pallas import tpu as pltpu
from jax.experimental.pallas import tpu_sc as plsc


_CHUNK_ROWS = 32
_NBUF = 3


def _make_sc_copy(seq, dim, dtype):
    info = plsc.get_sparse_core_info()
    nc, ns = info.num_cores, info.num_subcores
    nw = nc * ns
    rows_per_w = seq // nw
    chunk = _CHUNK_ROWS
    nbuf = _NBUF
    nchunk = rows_per_w // chunk
    mesh = plsc.VectorSubcoreMesh(core_axis_name="c", subcore_axis_name="s")

    @functools.partial(
        pl.kernel,
        mesh=mesh,
        out_type=jax.ShapeDtypeStruct((1, seq, dim), dtype),
        scratch_types=[
            pltpu.VMEM_SHARED((ns, nbuf, chunk, dim), dtype),
            pltpu.SemaphoreType.DMA((nbuf,)),
            pltpu.SemaphoreType.DMA((nbuf,)),
        ],
    )
    def sc_copy(w_hbm, out_hbm, buf, in_sems, out_sems):
        sid = lax.axis_index("s")
        wid = sid * nc + lax.axis_index("c")
        base = wid * rows_per_w

        def in_copy(i, slot):
            return pltpu.make_async_copy(
                w_hbm.at[pl.ds(base + i * chunk, chunk)],
                buf.at[sid, slot],
                in_sems.at[slot],
            )

        def out_copy(i, slot):
            return pltpu.make_async_copy(
                buf.at[sid, slot],
                out_hbm.at[0, pl.ds(base + i * chunk, chunk)],
                out_sems.at[slot],
            )

        for j in range(min(nbuf - 1, nchunk)):
            in_copy(j, j).start()
        for i in range(nchunk):
            slot = i % nbuf
            in_copy(i, slot).wait()
            out_copy(i, slot).start()
            k = i + nbuf - 1
            if k < nchunk:
                kslot = k % nbuf
                if k - nbuf >= 0:
                    out_copy(k - nbuf, kslot).wait()
                in_copy(k, kslot).start()
        for i in range(max(0, nchunk - nbuf), nchunk):
            out_copy(i, i % nbuf).wait()

    return sc_copy


def kernel(x, embedding_weight):
    seq = x.shape[1]
    dim = embedding_weight.shape[1]
    return _make_sc_copy(seq, dim, embedding_weight.dtype)(embedding_weight[:seq])


# speedup vs baseline: 1.0297x; 1.0297x over previous
import functools

import jax
import jax.numpy as jnp
from jax import lax
from jax.experimental import pallas as pl
from jax.experimental.pallas import tpu as pltpu
from jax.experimental.pallas import tpu_sc as plsc

_CHUNK_ROWS = 256
_NBUF = 4


def _make_sc_copy(seq, dim, dtype):
    info = plsc.get_sparse_core_info()
    nc = info.num_cores
    rows_per_w = seq // nc
    chunk = _CHUNK_ROWS
    nbuf = _NBUF
    nchunk = rows_per_w // chunk
    mesh = plsc.ScalarSubcoreMesh(axis_name="c", num_cores=nc)

    @functools.partial(
        pl.kernel,
        mesh=mesh,
        out_type=jax.ShapeDtypeStruct((1, seq, dim), dtype),
        scratch_types=[
            pltpu.VMEM_SHARED((nbuf, chunk, dim), dtype),
            pltpu.SemaphoreType.DMA((nbuf,)),
            pltpu.SemaphoreType.DMA((nbuf,)),
        ],
    )
    def sc_copy(w_hbm, out_hbm, buf, in_sems, out_sems):
        base = lax.axis_index("c") * rows_per_w

        def in_copy(i, slot):
            return pltpu.make_async_copy(
                w_hbm.at[pl.ds(base + i * chunk, chunk)],
                buf.at[slot],
                in_sems.at[slot],
            )

        def out_copy(i, slot):
            return pltpu.make_async_copy(
                buf.at[slot],
                out_hbm.at[0, pl.ds(base + i * chunk, chunk)],
                out_sems.at[slot],
            )

        for j in range(min(nbuf - 1, nchunk)):
            in_copy(j, j).start()
        for i in range(nchunk):
            slot = i % nbuf
            in_copy(i, slot).wait()
            out_copy(i, slot).start()
            k = i + nbuf - 1
            if k < nchunk:
                kslot = k % nbuf
                if k - nbuf >= 0:
                    out_copy(k - nbuf, kslot).wait()
                in_copy(k, kslot).start()
        for i in range(max(0, nchunk - nbuf), nchunk):
            out_copy(i, i % nbuf).wait()

    return sc_copy


def kernel(x, embedding_weight):
    seq = x.shape[1]
    dim = embedding_weight.shape[1]
    return _make_sc_copy(seq, dim, embedding_weight.dtype)(embedding_weight[:seq])


# speedup vs baseline: 1.0377x; 1.0078x over previous
import functools

import jax
import jax.numpy as jnp
from jax import lax
from jax.experimental import pallas as pl
from jax.experimental.pallas import tpu as pltpu
from jax.experimental.pallas import tpu_sc as plsc

_CHUNK_ROWS = 128
_NBUF = 8


def _make_sc_copy(seq, dim, dtype):
    info = plsc.get_sparse_core_info()
    nc = info.num_cores
    rows_per_w = seq // nc
    chunk = _CHUNK_ROWS
    nbuf = _NBUF
    nchunk = rows_per_w // chunk
    mesh = plsc.ScalarSubcoreMesh(axis_name="c", num_cores=nc)

    @functools.partial(
        pl.kernel,
        mesh=mesh,
        out_type=jax.ShapeDtypeStruct((1, seq, dim), dtype),
        scratch_types=[
            pltpu.VMEM_SHARED((nbuf, chunk, dim), dtype),
            pltpu.SemaphoreType.DMA((nbuf,)),
            pltpu.SemaphoreType.DMA((nbuf,)),
        ],
    )
    def sc_copy(w_hbm, out_hbm, buf, in_sems, out_sems):
        base = lax.axis_index("c") * rows_per_w

        def in_copy(i, slot):
            return pltpu.make_async_copy(
                w_hbm.at[pl.ds(base + i * chunk, chunk)],
                buf.at[slot],
                in_sems.at[slot],
            )

        def out_copy(i, slot):
            return pltpu.make_async_copy(
                buf.at[slot],
                out_hbm.at[0, pl.ds(base + i * chunk, chunk)],
                out_sems.at[slot],
            )

        for j in range(min(nbuf - 1, nchunk)):
            in_copy(j, j).start()
        for i in range(nchunk):
            slot = i % nbuf
            in_copy(i, slot).wait()
            out_copy(i, slot).start()
            k = i + nbuf - 1
            if k < nchunk:
                kslot = k % nbuf
                if k - nbuf >= 0:
                    out_copy(k - nbuf, kslot).wait()
                in_copy(k, kslot).start()
        for i in range(max(0, nchunk - nbuf), nchunk):
            out_copy(i, i % nbuf).wait()

    return sc_copy


def kernel(x, embedding_weight):
    seq = x.shape[1]
    dim = embedding_weight.shape[1]
    return _make_sc_copy(seq, dim, embedding_weight.dtype)(embedding_weight[:seq])
